# baseline (speedup 1.0000x reference)
import jax
import jax.numpy as jnp
from jax.experimental import pallas as pl
from jax.experimental.pallas import tpu as pltpu

N, F_IN, D = 10000, 256, 128
CIN = F_IN + D
BN = 1000


def _cell_body(x_ref, h_ref, wz_ref, bz_ref, wr_ref, br_ref, wh_ref, bh_ref,
               lw_ref, lb_ref, out_ref, H_ref, wbig_ref):
    @pl.when(pl.program_id(0) == 0)
    def _pack_weights():
        wbig_ref[:, 0:D] = (wz_ref[0, 0] + wz_ref[1, 0]).astype(jnp.bfloat16)
        wbig_ref[:, D:2 * D] = (wr_ref[0, 0] + wr_ref[1, 0]).astype(jnp.bfloat16)
        wbig_ref[:, 2 * D:3 * D] = (wh_ref[0, 0] + wh_ref[1, 0]).astype(jnp.bfloat16)

    hb = h_ref[...]
    xb = x_ref[...].astype(jnp.bfloat16)
    hb16 = hb.astype(jnp.bfloat16)

    def dot(a, b):
        return jax.lax.dot_general(a, b, (((1,), (0,)), ((), ())),
                                   preferred_element_type=jnp.float32)

    acc_x = dot(xb, wbig_ref[:F_IN, :])
    acc_h = dot(hb16, wbig_ref[F_IN:, :2 * D])
    z = jax.nn.sigmoid(acc_x[:, 0:D] + acc_h[:, 0:D] + bz_ref[...])
    r = jax.nn.sigmoid(acc_x[:, D:2 * D] + acc_h[:, D:2 * D] + br_ref[...])
    ht = jnp.tanh(acc_x[:, 2 * D:3 * D]
                  + dot((r * hb).astype(jnp.bfloat16), wbig_ref[F_IN:, 2 * D:3 * D])
                  + bh_ref[...])
    Hb = z * hb + (1.0 - z) * ht
    H_ref[...] = Hb
    relu = jnp.maximum(Hb, 0.0)
    out_ref[...] = jnp.sum(relu * lw_ref[...], axis=1, keepdims=True) + lb_ref[...]


@jax.jit
def _run(x, h, W_z, b_z, W_r, b_r, W_h, b_h, lin_w, lin_b):
    grid = (N // BN,)
    row_spec = lambda w: pl.BlockSpec((BN, w), lambda i: (i, 0))
    full_w = pl.BlockSpec((2, 1, CIN, D), lambda i: (0, 0, 0, 0))
    vec_spec = pl.BlockSpec((1, D), lambda i: (0, 0))
    out, H = pl.pallas_call(
        _cell_body,
        grid=grid,
        in_specs=[
            row_spec(F_IN),
            row_spec(D),
            full_w, vec_spec,
            full_w, vec_spec,
            full_w, vec_spec,
            vec_spec,
            pl.BlockSpec((1, 1), lambda i: (0, 0)),
        ],
        out_specs=[
            pl.BlockSpec((BN, 1), lambda i: (i, 0)),
            row_spec(D),
        ],
        out_shape=[
            jax.ShapeDtypeStruct((N, 1), jnp.float32),
            jax.ShapeDtypeStruct((N, D), jnp.float32),
        ],
        scratch_shapes=[pltpu.VMEM((CIN, 3 * D), jnp.bfloat16)],
    )(x, h, W_z, b_z.reshape(1, D), W_r, b_r.reshape(1, D),
      W_h, b_h.reshape(1, D), lin_w, lin_b.reshape(1, 1))
    return out, H


def kernel(x, edge_index, edge_attr, h, W_z, b_z, W_r, b_r, W_h, b_h,
           lin_w, lin_b):
    del edge_index, edge_attr
    return _run(x, h, W_z, b_z, W_r, b_r, W_h, b_h, lin_w, lin_b)


# speedup vs baseline: 1.1762x; 1.1762x over previous
import jax
import jax.numpy as jnp
from jax.experimental import pallas as pl
from jax.experimental.pallas import tpu as pltpu

N, F_IN, D = 10000, 256, 128
CIN = F_IN + D
BN = 5000


def _cell_body(x_ref, h_ref, wz_ref, bz_ref, wr_ref, br_ref, wh_ref, bh_ref,
               lw_ref, lb_ref, out_ref, H_ref, wbig_ref):
    @pl.when(pl.program_id(0) == 0)
    def _pack_weights():
        wbig_ref[:, 0:D] = (wz_ref[0, 0] + wz_ref[1, 0]).astype(jnp.bfloat16)
        wbig_ref[:, D:2 * D] = (wr_ref[0, 0] + wr_ref[1, 0]).astype(jnp.bfloat16)
        wbig_ref[:, 2 * D:3 * D] = (wh_ref[0, 0] + wh_ref[1, 0]).astype(jnp.bfloat16)

    hb = h_ref[...]
    xb = x_ref[...].astype(jnp.bfloat16)
    hb16 = hb.astype(jnp.bfloat16)

    def dot(a, b):
        return jax.lax.dot_general(a, b, (((1,), (0,)), ((), ())),
                                   preferred_element_type=jnp.float32)

    acc_x = dot(xb, wbig_ref[:F_IN, :])
    acc_h = dot(hb16, wbig_ref[F_IN:, :2 * D])
    z = jax.nn.sigmoid(acc_x[:, 0:D] + acc_h[:, 0:D] + bz_ref[...])
    r = jax.nn.sigmoid(acc_x[:, D:2 * D] + acc_h[:, D:2 * D] + br_ref[...])
    ht = jnp.tanh(acc_x[:, 2 * D:3 * D]
                  + dot((r * hb).astype(jnp.bfloat16), wbig_ref[F_IN:, 2 * D:3 * D])
                  + bh_ref[...])
    Hb = z * hb + (1.0 - z) * ht
    H_ref[...] = Hb
    relu = jnp.maximum(Hb, 0.0)
    out_ref[...] = jnp.sum(relu * lw_ref[...], axis=1, keepdims=True) + lb_ref[...]


@jax.jit
def _run(x, h, W_z, b_z, W_r, b_r, W_h, b_h, lin_w, lin_b):
    grid = (N // BN,)
    row_spec = lambda w: pl.BlockSpec((BN, w), lambda i: (i, 0))
    full_w = pl.BlockSpec((2, 1, CIN, D), lambda i: (0, 0, 0, 0))
    vec_spec = pl.BlockSpec((1, D), lambda i: (0, 0))
    out, H = pl.pallas_call(
        _cell_body,
        grid=grid,
        in_specs=[
            row_spec(F_IN),
            row_spec(D),
            full_w, vec_spec,
            full_w, vec_spec,
            full_w, vec_spec,
            vec_spec,
            pl.BlockSpec((1, 1), lambda i: (0, 0)),
        ],
        out_specs=[
            pl.BlockSpec((BN, 1), lambda i: (i, 0)),
            row_spec(D),
        ],
        out_shape=[
            jax.ShapeDtypeStruct((N, 1), jnp.float32),
            jax.ShapeDtypeStruct((N, D), jnp.float32),
        ],
        scratch_shapes=[pltpu.VMEM((CIN, 3 * D), jnp.bfloat16)],
    )(x, h, W_z, b_z.reshape(1, D), W_r, b_r.reshape(1, D),
      W_h, b_h.reshape(1, D), lin_w, lin_b.reshape(1, 1))
    return out, H


def kernel(x, edge_index, edge_attr, h, W_z, b_z, W_r, b_r, W_h, b_h,
           lin_w, lin_b):
    del edge_index, edge_attr
    return _run(x, h, W_z, b_z, W_r, b_r, W_h, b_h, lin_w, lin_b)
